# Initial kernel scaffold; baseline (speedup 1.0000x reference)
import jax
import jax.numpy as jnp
from jax.experimental import pallas as pl


def kernel(entity_emb, relation_emb, h_pos, r_pos, t_pos, h_neg, r_neg, t_neg):
    raise NotImplementedError("write your pallas kernel here")


# speedup vs baseline: 1.6937x; 1.6937x over previous
import functools

import jax
import jax.numpy as jnp
from jax import lax
from jax.experimental import pallas as pl
from jax.experimental.pallas import tpu as pltpu
from jax.experimental.pallas import tpu_sc as plsc

NC = 2
NS = 16
L = 16
NW = NC * NS

BATCH = 16384
D = 64
B_PER_W = BATCH // NW
CHUNK = 128
NCHUNK = B_PER_W // CHUNK
MARGIN = 1.0


def _transe_body(ent_hbm, rel_hbm, hp_hbm, rp_hbm, tp_hbm, hn_hbm, rn_hbm,
                 tn_hbm, out_hbm,
                 hp_i, rp_i, tp_i, hn_i, rn_i, tn_i,
                 hp_r, rp_r, tp_r, hn_r, rn_r, tn_r,
                 out_v, sem):
  wid = lax.axis_index("s") * NC + lax.axis_index("c")

  def chunk_total(c):
    pltpu.sync_copy(hp_hbm.at[wid, c], hp_i)
    pltpu.sync_copy(rp_hbm.at[wid, c], rp_i)
    pltpu.sync_copy(tp_hbm.at[wid, c], tp_i)
    pltpu.sync_copy(hn_hbm.at[wid, c], hn_i)
    pltpu.sync_copy(rn_hbm.at[wid, c], rn_i)
    pltpu.sync_copy(tn_hbm.at[wid, c], tn_i)
    copies = [
        pltpu.make_async_copy(ent_hbm.at[hp_i], hp_r, sem),
        pltpu.make_async_copy(rel_hbm.at[rp_i], rp_r, sem),
        pltpu.make_async_copy(ent_hbm.at[tp_i], tp_r, sem),
        pltpu.make_async_copy(ent_hbm.at[hn_i], hn_r, sem),
        pltpu.make_async_copy(rel_hbm.at[rn_i], rn_r, sem),
        pltpu.make_async_copy(ent_hbm.at[tn_i], tn_r, sem),
    ]
    for cp in copies:
      cp.start()
    for cp in copies:
      cp.wait()

    lane = lax.iota(jnp.int32, L)

    def body(t, tot):
      dv = jnp.zeros((L,), jnp.float32)
      for k in range(D // L):
        sl = pl.ds(k * L, L)
        dv += jnp.abs(hp_r[t, sl] + rp_r[t, sl] - tp_r[t, sl])
        dv -= jnp.abs(hn_r[t, sl] + rn_r[t, sl] - tn_r[t, sl])
      for shift in (1, 2, 4, 8):
        dv = dv + jnp.take_along_axis(dv, lane ^ shift, axis=0,
                                      mode="promise_in_bounds")
      return tot + jnp.maximum(dv + MARGIN, 0.0)

    return lax.fori_loop(0, CHUNK, body, jnp.zeros((L,), jnp.float32))

  total = jnp.zeros((L,), jnp.float32)
  for c in range(NCHUNK):
    total = total + chunk_total(c)

  lane = lax.iota(jnp.int32, L)
  out_v[...] = jnp.where(lane == 0, total, 0.0)
  pltpu.sync_copy(out_v, out_hbm.at[wid])


@jax.jit
def _transe_call(entity_emb, relation_emb, hp, rp, tp, hn, rn, tn):
  mesh = plsc.VectorSubcoreMesh(
      core_axis_name="c", subcore_axis_name="s", num_cores=NC,
      num_subcores=NS)
  idx_t = pltpu.VMEM((CHUNK,), jnp.int32)
  row_t = pltpu.VMEM((CHUNK, D), jnp.float32)
  grid_kernel = pl.kernel(
      _transe_body,
      out_type=jax.ShapeDtypeStruct((NW, L), jnp.float32),
      mesh=mesh,
      scratch_types=[
          idx_t, idx_t, idx_t, idx_t, idx_t, idx_t,
          row_t, row_t, row_t, row_t, row_t, row_t,
          pltpu.VMEM((L,), jnp.float32),
          pltpu.SemaphoreType.DMA,
      ],
      compiler_params=pltpu.CompilerParams(use_tc_tiling_on_sc=False),
  )
  partials = grid_kernel(entity_emb, relation_emb, hp, rp, tp, hn, rn, tn)
  return jnp.sum(partials) / BATCH


def kernel(entity_emb, relation_emb, h_pos, r_pos, t_pos, h_neg, r_neg,
           t_neg):
  shape = (NW, NCHUNK, CHUNK)
  return _transe_call(
      entity_emb, relation_emb,
      h_pos.astype(jnp.int32).reshape(shape),
      r_pos.astype(jnp.int32).reshape(shape),
      t_pos.astype(jnp.int32).reshape(shape),
      h_neg.astype(jnp.int32).reshape(shape),
      r_neg.astype(jnp.int32).reshape(shape),
      t_neg.astype(jnp.int32).reshape(shape),
  )


# speedup vs baseline: 1.8499x; 1.0922x over previous
import functools

import jax
import jax.numpy as jnp
from jax import lax
from jax.experimental import pallas as pl
from jax.experimental.pallas import tpu as pltpu
from jax.experimental.pallas import tpu_sc as plsc

NC = 2
NS = 16
L = 16
NW = NC * NS

BATCH = 16384
D = 64
B_PER_W = BATCH // NW
CHUNK = 128
NCHUNK = B_PER_W // CHUNK
MARGIN = 1.0


def _transe_body(ent_hbm, rel_hbm, hp_hbm, rp_hbm, tp_hbm, hn_hbm, rn_hbm,
                 tn_hbm, out_hbm, idx_v, row_v, out_v, sem):
  wid = lax.axis_index("s") * NC + lax.axis_index("c")

  def fire(c, buf):
    for a, src in enumerate((hp_hbm, rp_hbm, tp_hbm, hn_hbm, rn_hbm,
                             tn_hbm)):
      pltpu.sync_copy(src.at[wid, c], idx_v.at[buf, a])
    for a, table in enumerate((ent_hbm, rel_hbm, ent_hbm, ent_hbm, rel_hbm,
                               ent_hbm)):
      pltpu.make_async_copy(table.at[idx_v.at[buf, a]], row_v.at[buf, a],
                            sem.at[buf]).start()

  def drain(buf):
    for a, table in enumerate((ent_hbm, rel_hbm, ent_hbm, ent_hbm, rel_hbm,
                               ent_hbm)):
      pltpu.make_async_copy(table.at[idx_v.at[buf, a]], row_v.at[buf, a],
                            sem.at[buf]).wait()

  lane = lax.iota(jnp.int32, L)

  def compute(buf, total):
    def body(t, tot):
      dv = jnp.zeros((L,), jnp.float32)
      for k in range(D // L):
        sl = pl.ds(k * L, L)
        dv += jnp.abs(row_v[buf, 0, t, sl] + row_v[buf, 1, t, sl]
                      - row_v[buf, 2, t, sl])
        dv -= jnp.abs(row_v[buf, 3, t, sl] + row_v[buf, 4, t, sl]
                      - row_v[buf, 5, t, sl])
      for shift in (1, 2, 4, 8):
        dv = dv + jnp.take_along_axis(dv, lane ^ shift, axis=0,
                                      mode="promise_in_bounds")
      return tot + jnp.maximum(dv + MARGIN, 0.0)

    return lax.fori_loop(0, CHUNK, body, total)

  total = jnp.zeros((L,), jnp.float32)
  fire(0, 0)
  for c in range(1, NCHUNK):
    fire(c, c % 2)
    drain((c - 1) % 2)
    total = compute((c - 1) % 2, total)
  drain((NCHUNK - 1) % 2)
  total = compute((NCHUNK - 1) % 2, total)

  out_v[...] = jnp.where(lane == 0, total, 0.0)
  pltpu.sync_copy(out_v, out_hbm.at[wid])


@jax.jit
def _transe_call(entity_emb, relation_emb, hp, rp, tp, hn, rn, tn):
  mesh = plsc.VectorSubcoreMesh(
      core_axis_name="c", subcore_axis_name="s", num_cores=NC,
      num_subcores=NS)
  grid_kernel = pl.kernel(
      _transe_body,
      out_type=jax.ShapeDtypeStruct((NW, L), jnp.float32),
      mesh=mesh,
      scratch_types=[
          pltpu.VMEM((2, 6, CHUNK), jnp.int32),
          pltpu.VMEM((2, 6, CHUNK, D), jnp.float32),
          pltpu.VMEM((L,), jnp.float32),
          pltpu.SemaphoreType.DMA((2,)),
      ],
      compiler_params=pltpu.CompilerParams(use_tc_tiling_on_sc=False),
  )
  partials = grid_kernel(entity_emb, relation_emb, hp, rp, tp, hn, rn, tn)
  return jnp.sum(partials) / BATCH


def kernel(entity_emb, relation_emb, h_pos, r_pos, t_pos, h_neg, r_neg,
           t_neg):
  shape = (NW, NCHUNK, CHUNK)
  return _transe_call(
      entity_emb, relation_emb,
      h_pos.astype(jnp.int32).reshape(shape),
      r_pos.astype(jnp.int32).reshape(shape),
      t_pos.astype(jnp.int32).reshape(shape),
      h_neg.astype(jnp.int32).reshape(shape),
      r_neg.astype(jnp.int32).reshape(shape),
      t_neg.astype(jnp.int32).reshape(shape),
  )
